# baseline (speedup 1.0000x reference)
import jax
import jax.numpy as jnp
from jax import lax
from jax.experimental import pallas as pl
from jax.experimental.pallas import tpu as pltpu

B, N, D = 1024, 50, 512
NUM_TYPES = 4
CH = 16
NBUF = 8
LEAD = 4
T = B // CH


def _body(ids_ref, emb_ref, x_hbm, o_hbm, temb_v, *rest):
    bufs = rest[:NBUF]
    isems = rest[NBUF:2 * NBUF]
    osems = rest[2 * NBUF:3 * NBUF]

    tid = ids_ref[...]
    oh = (tid == lax.broadcasted_iota(jnp.int32, (N, NUM_TYPES), 1))
    temb_v[...] = jnp.dot(oh.astype(jnp.float32), emb_ref[...],
                          preferred_element_type=jnp.float32)

    def in_copy(t, p):
        return pltpu.make_async_copy(
            x_hbm.at[pl.ds(t * CH, CH)], bufs[p], isems[p])

    def out_copy(t, p):
        return pltpu.make_async_copy(
            bufs[p], o_hbm.at[pl.ds(t * CH, CH)], osems[p])

    for t in range(LEAD):
        in_copy(t, t % NBUF).start()

    temb = temb_v[...]

    def step(t0, carry):
        for p in range(NBUF):
            t = t0 + p
            in_copy(t, p).wait()

            @pl.when(t + LEAD < T)
            def _():
                pf = (p + LEAD) % NBUF

                @pl.when(t >= NBUF - LEAD)
                def _():
                    out_copy(t - (NBUF - LEAD), pf).wait()

                in_copy(t + LEAD, pf).start()

            bufs[p][...] = bufs[p][...] + temb[None]
            out_copy(t, p).start()
        return carry

    lax.fori_loop(0, T // NBUF, lambda s, c: step(s * NBUF, c), 0, unroll=False)

    for t in range(T - NBUF, T):
        out_copy(t, t % NBUF).wait()


def kernel(channel_stack, type_ids, embeddings):
    ids2 = type_ids.astype(jnp.int32).reshape(N, 1)
    return pl.pallas_call(
        _body,
        in_specs=[
            pl.BlockSpec(memory_space=pltpu.MemorySpace.VMEM),
            pl.BlockSpec(memory_space=pltpu.MemorySpace.VMEM),
            pl.BlockSpec(memory_space=pltpu.MemorySpace.HBM),
        ],
        out_specs=pl.BlockSpec(memory_space=pltpu.MemorySpace.HBM),
        out_shape=jax.ShapeDtypeStruct((B, N, D), jnp.float32),
        scratch_shapes=[pltpu.VMEM((N, D), jnp.float32)]
        + [pltpu.VMEM((CH, N, D), jnp.float32) for _ in range(NBUF)]
        + [pltpu.SemaphoreType.DMA for _ in range(2 * NBUF)],
    )(ids2, embeddings, channel_stack)
